# initial kernel scaffold (unmeasured)
import jax
import jax.numpy as jnp
from jax import lax
from jax.experimental import pallas as pl
from jax.experimental.pallas import tpu as pltpu

N_DEV = 8
M, K_SH, N = 4096, 512, 2048
CH = M // N_DEV


def _ring_perm(v):
    v = jnp.where(v == 4, 7, jnp.where(v == 7, 4, v))
    return jnp.where(v == 5, 6, jnp.where(v == 6, 5, v))


def _body(
    x_ref,
    w_ref,
    out_ref,
    part_ref,
    rs_buf,
    snd_buf,
    ag_buf,
    q_ref,
    amax_ref,
    rs_send,
    rs_recv,
    ag_send,
    ag_recv,
    ax_send,
    ax_recv,
    rs_credit,
    ag_credit,
):
    me = lax.axis_index("i").astype(jnp.int32)
    p = _ring_perm(me)
    nxt = _ring_perm((p + 1) % N_DEV)
    prv = _ring_perm((p - 1) % N_DEV)

    barrier = pltpu.get_barrier_semaphore()
    for nbr in (nxt, prv):
        pl.semaphore_signal(
            barrier, inc=1, device_id=(nbr,), device_id_type=pl.DeviceIdType.MESH
        )
    pl.semaphore_wait(barrier, 2)

    for j in range(N_DEV):
        c = (p - j) % N_DEV
        xs = x_ref[pl.ds(c * CH, CH), :]
        part_ref[j] = jnp.dot(
            xs, w_ref[:, :], preferred_element_type=jnp.float32
        ).astype(jnp.bfloat16)

    for s in range(N_DEV - 1):
        slot = s % 2
        if s == 0:
            src = part_ref.at[0]
        else:
            acc = rs_buf[(s - 1) % 2].astype(jnp.float32) + part_ref[s].astype(
                jnp.float32
            )
            snd_buf[slot] = acc.astype(jnp.bfloat16)
            src = snd_buf.at[slot]
        if s >= 2:
            pl.semaphore_wait(rs_credit, 1)
        rdma = pltpu.make_async_remote_copy(
            src_ref=src,
            dst_ref=rs_buf.at[slot],
            send_sem=rs_send.at[slot],
            recv_sem=rs_recv.at[slot],
            device_id=(nxt,),
            device_id_type=pl.DeviceIdType.MESH,
        )
        rdma.start()
        rdma.wait()
        if 1 <= s <= N_DEV - 3:
            pl.semaphore_signal(
                rs_credit,
                inc=1,
                device_id=(prv,),
                device_id_type=pl.DeviceIdType.MESH,
            )

    y = rs_buf[(N_DEV - 2) % 2].astype(jnp.float32) + part_ref[N_DEV - 1].astype(
        jnp.float32
    )
    y = jnp.maximum(y, 0.0)

    lmax = jnp.max(y)
    amax_ref[pl.ds(p, 1)] = jnp.full((1, 8, 128), lmax, dtype=jnp.float32)
    sends = []
    for o in range(1, N_DEV):
        tid = _ring_perm((p + o) % N_DEV)
        r = pltpu.make_async_remote_copy(
            src_ref=amax_ref.at[pl.ds(p, 1)],
            dst_ref=amax_ref.at[pl.ds(p, 1)],
            send_sem=ax_send.at[o - 1],
            recv_sem=ax_recv.at[o - 1],
            device_id=(tid,),
            device_id_type=pl.DeviceIdType.MESH,
        )
        r.start()
        sends.append(r)
    for o in range(1, N_DEV):
        sp = (p - o) % N_DEV
        r = pltpu.make_async_remote_copy(
            src_ref=amax_ref.at[pl.ds(p, 1)],
            dst_ref=amax_ref.at[pl.ds(sp, 1)],
            send_sem=ax_send.at[o - 1],
            recv_sem=ax_recv.at[o - 1],
            device_id=(nxt,),
            device_id_type=pl.DeviceIdType.MESH,
        )
        r.wait_recv()
    for r in sends:
        r.wait_send()

    gmax = jnp.max(amax_ref[:, :, :])
    scale = gmax / 127.0

    q_ref[:, :] = jnp.clip(jnp.round(y / scale), 0.0, 127.0).astype(jnp.int8)
    own = (p + 1) % N_DEV
    out_ref[pl.ds(own * CH, CH), :] = (
        q_ref[:, :].astype(jnp.float32) * scale
    ).astype(jnp.bfloat16)

    for t in range(N_DEV - 1):
        slot = t % 2
        src = q_ref if t == 0 else ag_buf.at[(t - 1) % 2]
        if t >= 2:
            pl.semaphore_wait(ag_credit, 1)
        rdma = pltpu.make_async_remote_copy(
            src_ref=src,
            dst_ref=ag_buf.at[slot],
            send_sem=ag_send.at[slot],
            recv_sem=ag_recv.at[slot],
            device_id=(nxt,),
            device_id_type=pl.DeviceIdType.MESH,
        )
        rdma.start()
        rdma.wait()
        c = (p - t) % N_DEV
        out_ref[pl.ds(c * CH, CH), :] = (
            ag_buf[slot].astype(jnp.float32) * scale
        ).astype(jnp.bfloat16)
        if 1 <= t <= N_DEV - 3:
            pl.semaphore_signal(
                ag_credit,
                inc=1,
                device_id=(prv,),
                device_id_type=pl.DeviceIdType.MESH,
            )


def kernel(x, w_mat):
    return pl.pallas_call(
        _body,
        out_shape=jax.ShapeDtypeStruct((M, N), jnp.bfloat16),
        in_specs=[
            pl.BlockSpec(memory_space=pltpu.VMEM),
            pl.BlockSpec(memory_space=pltpu.VMEM),
        ],
        out_specs=pl.BlockSpec(memory_space=pltpu.VMEM),
        scratch_shapes=[
            pltpu.VMEM((N_DEV, CH, N), jnp.bfloat16),
            pltpu.VMEM((2, CH, N), jnp.bfloat16),
            pltpu.VMEM((2, CH, N), jnp.bfloat16),
            pltpu.VMEM((2, CH, N), jnp.int8),
            pltpu.VMEM((CH, N), jnp.int8),
            pltpu.VMEM((N_DEV, 8, 128), jnp.float32),
            pltpu.SemaphoreType.DMA((2,)),
            pltpu.SemaphoreType.DMA((2,)),
            pltpu.SemaphoreType.DMA((2,)),
            pltpu.SemaphoreType.DMA((2,)),
            pltpu.SemaphoreType.DMA((N_DEV - 1,)),
            pltpu.SemaphoreType.DMA((N_DEV - 1,)),
            pltpu.SemaphoreType.REGULAR,
            pltpu.SemaphoreType.REGULAR,
        ],
        compiler_params=pltpu.CompilerParams(collective_id=0),
    )(x, w_mat)


# baseline (device time: 321910 ns/iter reference)
import jax
import jax.numpy as jnp
from jax import lax
from jax.experimental import pallas as pl
from jax.experimental.pallas import tpu as pltpu

N_DEV = 8
M, K_SH, N = 4096, 512, 2048
CH = M // N_DEV


def _ring_perm(v):
    v = jnp.where(v == 4, 7, jnp.where(v == 7, 4, v))
    return jnp.where(v == 5, 6, jnp.where(v == 6, 5, v))


def _body(
    x_ref,
    w_ref,
    out_ref,
    part_ref,
    rs_buf,
    snd_buf,
    ag_buf,
    q_ref,
    amax_ref,
    rs_send,
    rs_recv,
    ag_send,
    ag_recv,
    ax_send,
    ax_recv,
    rs_credit,
    ag_credit,
):
    me = lax.axis_index("i").astype(jnp.int32)
    p = _ring_perm(me)
    nxt = _ring_perm((p + 1) % N_DEV)
    prv = _ring_perm((p - 1) % N_DEV)

    barrier = pltpu.get_barrier_semaphore()
    for nbr in (nxt, prv):
        pl.semaphore_signal(
            barrier, inc=1, device_id=(nbr,), device_id_type=pl.DeviceIdType.MESH
        )
    pl.semaphore_wait(barrier, 2)

    for j in range(N_DEV):
        c = (p - j) % N_DEV
        xs = x_ref[pl.ds(c * CH, CH), :]
        part_ref[j] = jnp.dot(
            xs, w_ref[:, :], preferred_element_type=jnp.float32
        ).astype(jnp.bfloat16)

    for s in range(N_DEV - 1):
        slot = s % 2
        if s == 0:
            src = part_ref.at[0]
        else:
            acc = rs_buf[(s - 1) % 2].astype(jnp.float32) + part_ref[s].astype(
                jnp.float32
            )
            snd_buf[slot] = acc.astype(jnp.bfloat16)
            src = snd_buf.at[slot]
        if s >= 2:
            pl.semaphore_wait(rs_credit, 1)
        rdma = pltpu.make_async_remote_copy(
            src_ref=src,
            dst_ref=rs_buf.at[slot],
            send_sem=rs_send.at[slot],
            recv_sem=rs_recv.at[slot],
            device_id=(nxt,),
            device_id_type=pl.DeviceIdType.MESH,
        )
        rdma.start()
        rdma.wait()
        if 1 <= s <= N_DEV - 3:
            pl.semaphore_signal(
                rs_credit,
                inc=1,
                device_id=(prv,),
                device_id_type=pl.DeviceIdType.MESH,
            )

    y = rs_buf[(N_DEV - 2) % 2].astype(jnp.float32) + part_ref[N_DEV - 1].astype(
        jnp.float32
    )
    y = jnp.maximum(y, 0.0)

    lmax = jnp.max(y)
    amax_ref[pl.ds(p, 1)] = jnp.full((1, 8, 128), lmax, dtype=jnp.float32)
    sends = []
    for o in range(1, N_DEV):
        tid = _ring_perm((p + o) % N_DEV)
        r = pltpu.make_async_remote_copy(
            src_ref=amax_ref.at[pl.ds(p, 1)],
            dst_ref=amax_ref.at[pl.ds(p, 1)],
            send_sem=ax_send.at[o - 1],
            recv_sem=ax_recv.at[o - 1],
            device_id=(tid,),
            device_id_type=pl.DeviceIdType.MESH,
        )
        r.start()
        sends.append(r)
    for o in range(1, N_DEV):
        sp = (p - o) % N_DEV
        r = pltpu.make_async_remote_copy(
            src_ref=amax_ref.at[pl.ds(p, 1)],
            dst_ref=amax_ref.at[pl.ds(sp, 1)],
            send_sem=ax_send.at[o - 1],
            recv_sem=ax_recv.at[o - 1],
            device_id=(nxt,),
            device_id_type=pl.DeviceIdType.MESH,
        )
        r.wait_recv()
    for r in sends:
        r.wait_send()

    gmax = jnp.max(amax_ref[:, :, :])
    scale = gmax / 127.0

    q_ref[:, :] = jnp.clip(jnp.round(y / scale), 0.0, 127.0).astype(jnp.int8)
    own = (p + 1) % N_DEV
    out_ref[pl.ds(own * CH, CH), :] = (
        q_ref[:, :].astype(jnp.float32) * scale
    ).astype(jnp.bfloat16)

    for t in range(N_DEV - 1):
        slot = t % 2
        src = q_ref if t == 0 else ag_buf.at[(t - 1) % 2]
        if t >= 2:
            pl.semaphore_wait(ag_credit, 1)
        rdma = pltpu.make_async_remote_copy(
            src_ref=src,
            dst_ref=ag_buf.at[slot],
            send_sem=ag_send.at[slot],
            recv_sem=ag_recv.at[slot],
            device_id=(nxt,),
            device_id_type=pl.DeviceIdType.MESH,
        )
        rdma.start()
        rdma.wait()
        c = (p - t) % N_DEV
        out_ref[pl.ds(c * CH, CH), :] = (
            ag_buf[slot].astype(jnp.float32) * scale
        ).astype(jnp.bfloat16)
        if 1 <= t <= N_DEV - 3:
            pl.semaphore_signal(
                ag_credit,
                inc=1,
                device_id=(prv,),
                device_id_type=pl.DeviceIdType.MESH,
            )


def kernel(x, w_mat):
    x = x.astype(jnp.bfloat16)
    w_mat = w_mat.astype(jnp.bfloat16)
    return pl.pallas_call(
        _body,
        out_shape=jax.ShapeDtypeStruct((M, N), jnp.bfloat16),
        in_specs=[
            pl.BlockSpec(memory_space=pltpu.VMEM),
            pl.BlockSpec(memory_space=pltpu.VMEM),
        ],
        out_specs=pl.BlockSpec(memory_space=pltpu.VMEM),
        scratch_shapes=[
            pltpu.VMEM((N_DEV, CH, N), jnp.bfloat16),
            pltpu.VMEM((2, CH, N), jnp.bfloat16),
            pltpu.VMEM((2, CH, N), jnp.bfloat16),
            pltpu.VMEM((2, CH, N), jnp.int8),
            pltpu.VMEM((CH, N), jnp.int8),
            pltpu.VMEM((N_DEV, 8, 128), jnp.float32),
            pltpu.SemaphoreType.DMA((2,)),
            pltpu.SemaphoreType.DMA((2,)),
            pltpu.SemaphoreType.DMA((2,)),
            pltpu.SemaphoreType.DMA((2,)),
            pltpu.SemaphoreType.DMA((N_DEV - 1,)),
            pltpu.SemaphoreType.DMA((N_DEV - 1,)),
            pltpu.SemaphoreType.REGULAR,
            pltpu.SemaphoreType.REGULAR,
        ],
        compiler_params=pltpu.CompilerParams(
            collective_id=0, vmem_limit_bytes=60 * 1024 * 1024
        ),
    )(x, w_mat)


# device time: 204850 ns/iter; 1.5714x vs baseline; 1.5714x over previous
import jax
import jax.numpy as jnp
from jax import lax
from jax.experimental import pallas as pl
from jax.experimental.pallas import tpu as pltpu

N_DEV = 8
M, K_SH, N = 4096, 512, 2048
CH = M // N_DEV
NH = N // 2


def _ring_perm(v):
    v = jnp.where(v == 4, 7, jnp.where(v == 7, 4, v))
    return jnp.where(v == 5, 6, jnp.where(v == 6, 5, v))


def _body(
    x_ref,
    w_ref,
    out_ref,
    part_ref,
    rsf_buf,
    rsb_buf,
    sndf_buf,
    sndb_buf,
    agf_buf,
    agb_buf,
    qf_ref,
    qb_ref,
    amax_ref,
    rsf_send,
    rsf_recv,
    rsb_send,
    rsb_recv,
    agf_send,
    agf_recv,
    agb_send,
    agb_recv,
    ax_send,
    ax_recv,
    rsf_credit,
    rsb_credit,
    agf_credit,
    agb_credit,
):
    me = lax.axis_index("i").astype(jnp.int32)
    p = _ring_perm(me)
    nxt = _ring_perm((p + 1) % N_DEV)
    prv = _ring_perm((p - 1) % N_DEV)

    barrier = pltpu.get_barrier_semaphore()
    for nbr in (nxt, prv):
        pl.semaphore_signal(
            barrier, inc=1, device_id=(nbr,), device_id_type=pl.DeviceIdType.MESH
        )
    pl.semaphore_wait(barrier, 2)

    for j in range(N_DEV):
        c = (p - j) % N_DEV
        xs = x_ref[pl.ds(c * CH, CH), :]
        part_ref[j] = jnp.dot(
            xs, w_ref[:, :], preferred_element_type=jnp.float32
        ).astype(jnp.bfloat16)

    for s in range(N_DEV - 1):
        slot = s % 2
        if s == 0:
            sndf_buf[0] = part_ref[0, :, 0:NH]
            sndb_buf[0] = part_ref[0, :, NH:N]
        else:
            accf = rsf_buf[(s - 1) % 2].astype(jnp.float32) + part_ref[
                s, :, 0:NH
            ].astype(jnp.float32)
            sndf_buf[slot] = accf.astype(jnp.bfloat16)
            accb = rsb_buf[(s - 1) % 2].astype(jnp.float32) + part_ref[
                (N_DEV - s) % N_DEV, :, NH:N
            ].astype(jnp.float32)
            sndb_buf[slot] = accb.astype(jnp.bfloat16)
        if s >= 2:
            pl.semaphore_wait(rsf_credit, 1)
            pl.semaphore_wait(rsb_credit, 1)
        rdma_f = pltpu.make_async_remote_copy(
            src_ref=sndf_buf.at[slot],
            dst_ref=rsf_buf.at[slot],
            send_sem=rsf_send.at[slot],
            recv_sem=rsf_recv.at[slot],
            device_id=(nxt,),
            device_id_type=pl.DeviceIdType.MESH,
        )
        rdma_b = pltpu.make_async_remote_copy(
            src_ref=sndb_buf.at[slot],
            dst_ref=rsb_buf.at[slot],
            send_sem=rsb_send.at[slot],
            recv_sem=rsb_recv.at[slot],
            device_id=(prv,),
            device_id_type=pl.DeviceIdType.MESH,
        )
        rdma_f.start()
        rdma_b.start()
        rdma_f.wait()
        rdma_b.wait()
        if 1 <= s <= N_DEV - 3:
            pl.semaphore_signal(
                rsf_credit,
                inc=1,
                device_id=(prv,),
                device_id_type=pl.DeviceIdType.MESH,
            )
            pl.semaphore_signal(
                rsb_credit,
                inc=1,
                device_id=(nxt,),
                device_id_type=pl.DeviceIdType.MESH,
            )

    y_f = rsf_buf[(N_DEV - 2) % 2].astype(jnp.float32) + part_ref[
        N_DEV - 1, :, 0:NH
    ].astype(jnp.float32)
    y_f = jnp.maximum(y_f, 0.0)
    y_b = rsb_buf[(N_DEV - 2) % 2].astype(jnp.float32) + part_ref[
        1, :, NH:N
    ].astype(jnp.float32)
    y_b = jnp.maximum(y_b, 0.0)

    lmax = jnp.maximum(jnp.max(y_f), jnp.max(y_b))
    amax_ref[pl.ds(p, 1)] = jnp.full((1, 8, 128), lmax, dtype=jnp.float32)
    sends = []
    for o in range(1, N_DEV):
        tid = _ring_perm((p + o) % N_DEV)
        r = pltpu.make_async_remote_copy(
            src_ref=amax_ref.at[pl.ds(p, 1)],
            dst_ref=amax_ref.at[pl.ds(p, 1)],
            send_sem=ax_send.at[o - 1],
            recv_sem=ax_recv.at[o - 1],
            device_id=(tid,),
            device_id_type=pl.DeviceIdType.MESH,
        )
        r.start()
        sends.append(r)
    for o in range(1, N_DEV):
        sp = (p - o) % N_DEV
        r = pltpu.make_async_remote_copy(
            src_ref=amax_ref.at[pl.ds(p, 1)],
            dst_ref=amax_ref.at[pl.ds(sp, 1)],
            send_sem=ax_send.at[o - 1],
            recv_sem=ax_recv.at[o - 1],
            device_id=(nxt,),
            device_id_type=pl.DeviceIdType.MESH,
        )
        r.wait_recv()
    for r in sends:
        r.wait_send()

    gmax = jnp.max(amax_ref[:, :, :])
    scale = gmax / 127.0

    qf_ref[:, :] = jnp.clip(jnp.round(y_f / scale), 0.0, 127.0).astype(jnp.int8)
    qb_ref[:, :] = jnp.clip(jnp.round(y_b / scale), 0.0, 127.0).astype(jnp.int8)
    own_f = (p + 1) % N_DEV
    own_b = (p - 1) % N_DEV
    out_ref[pl.ds(own_f * CH, CH), 0:NH] = (
        qf_ref[:, :].astype(jnp.float32) * scale
    ).astype(jnp.bfloat16)
    out_ref[pl.ds(own_b * CH, CH), NH:N] = (
        qb_ref[:, :].astype(jnp.float32) * scale
    ).astype(jnp.bfloat16)

    for t in range(N_DEV - 1):
        slot = t % 2
        src_f = qf_ref if t == 0 else agf_buf.at[(t - 1) % 2]
        src_b = qb_ref if t == 0 else agb_buf.at[(t - 1) % 2]
        if t >= 2:
            pl.semaphore_wait(agf_credit, 1)
            pl.semaphore_wait(agb_credit, 1)
        rdma_f = pltpu.make_async_remote_copy(
            src_ref=src_f,
            dst_ref=agf_buf.at[slot],
            send_sem=agf_send.at[slot],
            recv_sem=agf_recv.at[slot],
            device_id=(nxt,),
            device_id_type=pl.DeviceIdType.MESH,
        )
        rdma_b = pltpu.make_async_remote_copy(
            src_ref=src_b,
            dst_ref=agb_buf.at[slot],
            send_sem=agb_send.at[slot],
            recv_sem=agb_recv.at[slot],
            device_id=(prv,),
            device_id_type=pl.DeviceIdType.MESH,
        )
        rdma_f.start()
        rdma_b.start()
        rdma_f.wait()
        rdma_b.wait()
        cf = (p - t) % N_DEV
        cb = (p + t) % N_DEV
        out_ref[pl.ds(cf * CH, CH), 0:NH] = (
            agf_buf[slot].astype(jnp.float32) * scale
        ).astype(jnp.bfloat16)
        out_ref[pl.ds(cb * CH, CH), NH:N] = (
            agb_buf[slot].astype(jnp.float32) * scale
        ).astype(jnp.bfloat16)
        if 1 <= t <= N_DEV - 3:
            pl.semaphore_signal(
                agf_credit,
                inc=1,
                device_id=(prv,),
                device_id_type=pl.DeviceIdType.MESH,
            )
            pl.semaphore_signal(
                agb_credit,
                inc=1,
                device_id=(nxt,),
                device_id_type=pl.DeviceIdType.MESH,
            )


def kernel(x, w_mat):
    x = x.astype(jnp.bfloat16)
    w_mat = w_mat.astype(jnp.bfloat16)
    return pl.pallas_call(
        _body,
        out_shape=jax.ShapeDtypeStruct((M, N), jnp.bfloat16),
        in_specs=[
            pl.BlockSpec(memory_space=pltpu.VMEM),
            pl.BlockSpec(memory_space=pltpu.VMEM),
        ],
        out_specs=pl.BlockSpec(memory_space=pltpu.VMEM),
        scratch_shapes=[
            pltpu.VMEM((N_DEV, CH, N), jnp.bfloat16),
            pltpu.VMEM((2, CH, NH), jnp.bfloat16),
            pltpu.VMEM((2, CH, NH), jnp.bfloat16),
            pltpu.VMEM((2, CH, NH), jnp.bfloat16),
            pltpu.VMEM((2, CH, NH), jnp.bfloat16),
            pltpu.VMEM((2, CH, NH), jnp.int8),
            pltpu.VMEM((2, CH, NH), jnp.int8),
            pltpu.VMEM((CH, NH), jnp.int8),
            pltpu.VMEM((CH, NH), jnp.int8),
            pltpu.VMEM((N_DEV, 8, 128), jnp.float32),
            pltpu.SemaphoreType.DMA((2,)),
            pltpu.SemaphoreType.DMA((2,)),
            pltpu.SemaphoreType.DMA((2,)),
            pltpu.SemaphoreType.DMA((2,)),
            pltpu.SemaphoreType.DMA((2,)),
            pltpu.SemaphoreType.DMA((2,)),
            pltpu.SemaphoreType.DMA((2,)),
            pltpu.SemaphoreType.DMA((2,)),
            pltpu.SemaphoreType.DMA((N_DEV - 1,)),
            pltpu.SemaphoreType.DMA((N_DEV - 1,)),
            pltpu.SemaphoreType.REGULAR,
            pltpu.SemaphoreType.REGULAR,
            pltpu.SemaphoreType.REGULAR,
            pltpu.SemaphoreType.REGULAR,
        ],
        compiler_params=pltpu.CompilerParams(
            collective_id=0, vmem_limit_bytes=60 * 1024 * 1024
        ),
    )(x, w_mat)


# device time: 168325 ns/iter; 1.9124x vs baseline; 1.2170x over previous
import jax
import jax.numpy as jnp
from jax import lax
from jax.experimental import pallas as pl
from jax.experimental.pallas import tpu as pltpu

N_DEV = 8
M, K_SH, N = 4096, 512, 2048
CH = M // N_DEV
NH = N // 2
NQ = NH // 2


def _ring_perm(v):
    v = jnp.where(v == 4, 7, jnp.where(v == 7, 4, v))
    return jnp.where(v == 5, 6, jnp.where(v == 6, 5, v))


def _body(
    x_ref,
    w_ref,
    out_ref,
    part_ref,
    rsf_buf,
    rsb_buf,
    sndf_buf,
    sndb_buf,
    agf_buf,
    agb_buf,
    qf_ref,
    qb_ref,
    amax_ref,
    rsf_send,
    rsf_recv,
    rsb_send,
    rsb_recv,
    agf_send,
    agf_recv,
    agb_send,
    agb_recv,
    ax_send,
    ax_recv,
    rsf_credit,
    rsb_credit,
    agf_credit,
    agb_credit,
):
    me = lax.axis_index("i").astype(jnp.int32)
    p = _ring_perm(me)
    nxt = _ring_perm((p + 1) % N_DEV)
    prv = _ring_perm((p - 1) % N_DEV)

    def gemm_chunk(j):
        c = (p - j) % N_DEV
        part_ref[j] = jnp.dot(
            x_ref[pl.ds(c * CH, CH), :], w_ref[:, :],
            preferred_element_type=jnp.float32,
        ).astype(jnp.bfloat16)

    def rs_desc(dirn, slot, q):
        snd, rcv, ssem, rsem, tgt = (
            (sndf_buf, rsf_buf, rsf_send, rsf_recv, nxt)
            if dirn == 0
            else (sndb_buf, rsb_buf, rsb_send, rsb_recv, prv)
        )
        return pltpu.make_async_remote_copy(
            src_ref=snd.at[slot, q],
            dst_ref=rcv.at[slot, q],
            send_sem=ssem.at[slot, q],
            recv_sem=rsem.at[slot, q],
            device_id=(tgt,),
            device_id_type=pl.DeviceIdType.MESH,
        )

    def ag_desc(dirn, slot, q, t0_src=False):
        buf, ssem, rsem, tgt = (
            (agf_buf, agf_send, agf_recv, nxt)
            if dirn == 0
            else (agb_buf, agb_send, agb_recv, prv)
        )
        src = (qf_ref if dirn == 0 else qb_ref).at[q] if t0_src else buf.at[
            (slot + 1) % 2, q
        ]
        return pltpu.make_async_remote_copy(
            src_ref=src,
            dst_ref=buf.at[slot, q],
            send_sem=ssem.at[slot, q],
            recv_sem=rsem.at[slot, q],
            device_id=(tgt,),
            device_id_type=pl.DeviceIdType.MESH,
        )

    def credit_sig(sem, tgt):
        pl.semaphore_signal(
            sem, inc=1, device_id=(tgt,), device_id_type=pl.DeviceIdType.MESH
        )

    gemm_chunk(0)

    barrier = pltpu.get_barrier_semaphore()
    for nbr in (nxt, prv):
        pl.semaphore_signal(
            barrier, inc=1, device_id=(nbr,), device_id_type=pl.DeviceIdType.MESH
        )
    pl.semaphore_wait(barrier, 2)

    computed = {0}
    for s in range(N_DEV - 1):
        slot, pslot = s % 2, (s - 1) % 2
        for q in range(2):
            for dirn in range(2):
                snd = sndf_buf if dirn == 0 else sndb_buf
                off = q * NQ if dirn == 0 else NH + q * NQ
                if s >= 1:
                    rs_desc(dirn, pslot, q).wait_recv()
                if s >= 2:
                    rs_desc(dirn, slot, q).wait_send()
                if s == 0:
                    snd[slot, q] = part_ref[0, :, off : off + NQ]
                else:
                    pj = s if dirn == 0 else (N_DEV - s) % N_DEV
                    rcv = rsf_buf if dirn == 0 else rsb_buf
                    acc = rcv[pslot, q].astype(jnp.float32) + part_ref[
                        pj, :, off : off + NQ
                    ].astype(jnp.float32)
                    snd[slot, q] = acc.astype(jnp.bfloat16)
                if s >= 2:
                    pl.semaphore_wait(rsf_credit if dirn == 0 else rsb_credit, 1)
                rs_desc(dirn, slot, q).start()
                if 1 <= s <= N_DEV - 3:
                    credit_sig(
                        rsf_credit if dirn == 0 else rsb_credit,
                        prv if dirn == 0 else nxt,
                    )
        for cj in (s + 1, (N_DEV - (s + 1)) % N_DEV):
            if 0 < cj < N_DEV and cj not in computed:
                computed.add(cj)
                gemm_chunk(cj)

    ys = {}
    for q in range(2):
        for dirn in range(2):
            rs_desc(dirn, 0, q).wait_recv()
            pj = N_DEV - 1 if dirn == 0 else 1
            off = q * NQ if dirn == 0 else NH + q * NQ
            rcv = rsf_buf if dirn == 0 else rsb_buf
            y = rcv[0, q].astype(jnp.float32) + part_ref[
                pj, :, off : off + NQ
            ].astype(jnp.float32)
            ys[(dirn, q)] = jnp.maximum(y, 0.0)

    lmax = jnp.maximum(
        jnp.maximum(jnp.max(ys[(0, 0)]), jnp.max(ys[(0, 1)])),
        jnp.maximum(jnp.max(ys[(1, 0)]), jnp.max(ys[(1, 1)])),
    )
    amax_ref[pl.ds(p, 1)] = jnp.full((1, 8, 128), lmax, dtype=jnp.float32)
    ax_sends = []
    for o in range(1, N_DEV):
        tid = _ring_perm((p + o) % N_DEV)
        r = pltpu.make_async_remote_copy(
            src_ref=amax_ref.at[pl.ds(p, 1)],
            dst_ref=amax_ref.at[pl.ds(p, 1)],
            send_sem=ax_send.at[o - 1],
            recv_sem=ax_recv.at[o - 1],
            device_id=(tid,),
            device_id_type=pl.DeviceIdType.MESH,
        )
        r.start()
        ax_sends.append(r)
    for slot_d in (1, 0):
        for q in range(2):
            for dirn in range(2):
                rs_desc(dirn, slot_d, q).wait_send()
    for o in range(1, N_DEV):
        sp = (p - o) % N_DEV
        r = pltpu.make_async_remote_copy(
            src_ref=amax_ref.at[pl.ds(p, 1)],
            dst_ref=amax_ref.at[pl.ds(sp, 1)],
            send_sem=ax_send.at[o - 1],
            recv_sem=ax_recv.at[o - 1],
            device_id=(nxt,),
            device_id_type=pl.DeviceIdType.MESH,
        )
        r.wait_recv()
    for r in ax_sends:
        r.wait_send()

    gmax = jnp.max(amax_ref[:, :, :])
    scale = gmax / 127.0

    own_f = (p + 1) % N_DEV
    own_b = (p - 1) % N_DEV
    for q in range(2):
        qf_ref[q] = jnp.clip(jnp.round(ys[(0, q)] / scale), 0.0, 127.0).astype(
            jnp.int8
        )
        qb_ref[q] = jnp.clip(jnp.round(ys[(1, q)] / scale), 0.0, 127.0).astype(
            jnp.int8
        )
        out_ref[pl.ds(own_f * CH, CH), q * NQ : (q + 1) * NQ] = (
            qf_ref[q].astype(jnp.float32) * scale
        ).astype(jnp.bfloat16)
        out_ref[pl.ds(own_b * CH, CH), NH + q * NQ : NH + (q + 1) * NQ] = (
            qb_ref[q].astype(jnp.float32) * scale
        ).astype(jnp.bfloat16)

    def deq_store(dirn, slot, q, t):
        c = (p - t) % N_DEV if dirn == 0 else (p + t) % N_DEV
        off = q * NQ if dirn == 0 else NH + q * NQ
        buf = agf_buf if dirn == 0 else agb_buf
        out_ref[pl.ds(c * CH, CH), off : off + NQ] = (
            buf[slot, q].astype(jnp.float32) * scale
        ).astype(jnp.bfloat16)

    for t in range(N_DEV - 1):
        slot, pslot = t % 2, (t - 1) % 2
        descs = []
        for q in range(2):
            for dirn in range(2):
                if t >= 1:
                    ag_desc(dirn, pslot, q).wait_recv()
                if t >= 2:
                    pl.semaphore_wait(agf_credit if dirn == 0 else agb_credit, 1)
                d = ag_desc(dirn, slot, q, t0_src=(t == 0))
                d.start()
                descs.append((d, dirn, q))
        for d, dirn, q in descs:
            if t >= 1:
                deq_store(dirn, pslot, q, t - 1)
            d.wait_send()
            if 1 <= t <= N_DEV - 3:
                credit_sig(
                    agf_credit if dirn == 0 else agb_credit,
                    prv if dirn == 0 else nxt,
                )
    for q in range(2):
        for dirn in range(2):
            ag_desc(dirn, 0, q).wait_recv()
            deq_store(dirn, 0, q, N_DEV - 2)


def kernel(x, w_mat):
    x = x.astype(jnp.bfloat16)
    w_mat = w_mat.astype(jnp.bfloat16)
    return pl.pallas_call(
        _body,
        out_shape=jax.ShapeDtypeStruct((M, N), jnp.bfloat16),
        in_specs=[
            pl.BlockSpec(memory_space=pltpu.VMEM),
            pl.BlockSpec(memory_space=pltpu.VMEM),
        ],
        out_specs=pl.BlockSpec(memory_space=pltpu.VMEM),
        scratch_shapes=[
            pltpu.VMEM((N_DEV, CH, N), jnp.bfloat16),
            pltpu.VMEM((2, 2, CH, NQ), jnp.bfloat16),
            pltpu.VMEM((2, 2, CH, NQ), jnp.bfloat16),
            pltpu.VMEM((2, 2, CH, NQ), jnp.bfloat16),
            pltpu.VMEM((2, 2, CH, NQ), jnp.bfloat16),
            pltpu.VMEM((2, 2, CH, NQ), jnp.int8),
            pltpu.VMEM((2, 2, CH, NQ), jnp.int8),
            pltpu.VMEM((2, CH, NQ), jnp.int8),
            pltpu.VMEM((2, CH, NQ), jnp.int8),
            pltpu.VMEM((N_DEV, 8, 128), jnp.float32),
            pltpu.SemaphoreType.DMA((2, 2)),
            pltpu.SemaphoreType.DMA((2, 2)),
            pltpu.SemaphoreType.DMA((2, 2)),
            pltpu.SemaphoreType.DMA((2, 2)),
            pltpu.SemaphoreType.DMA((2, 2)),
            pltpu.SemaphoreType.DMA((2, 2)),
            pltpu.SemaphoreType.DMA((2, 2)),
            pltpu.SemaphoreType.DMA((2, 2)),
            pltpu.SemaphoreType.DMA((N_DEV - 1,)),
            pltpu.SemaphoreType.DMA((N_DEV - 1,)),
            pltpu.SemaphoreType.REGULAR,
            pltpu.SemaphoreType.REGULAR,
            pltpu.SemaphoreType.REGULAR,
            pltpu.SemaphoreType.REGULAR,
        ],
        compiler_params=pltpu.CompilerParams(
            collective_id=0, vmem_limit_bytes=60 * 1024 * 1024
        ),
    )(x, w_mat)


# device time: 162937 ns/iter; 1.9757x vs baseline; 1.0331x over previous
import jax
import jax.numpy as jnp
from jax import lax
from jax.experimental import pallas as pl
from jax.experimental.pallas import tpu as pltpu

N_DEV = 8
M, K_SH, N = 4096, 512, 2048
CH = M // N_DEV
NH = N // 2
NQ = NH // 2


def _ring_perm(v):
    v = jnp.where(v == 4, 7, jnp.where(v == 7, 4, v))
    return jnp.where(v == 5, 6, jnp.where(v == 6, 5, v))


def _body(
    x_ref,
    w_ref,
    out_ref,
    part_ref,
    rsf_buf,
    rsb_buf,
    sndf_buf,
    sndb_buf,
    agf_buf,
    agb_buf,
    qf_ref,
    qb_ref,
    amax_ref,
    rsf_send,
    rsf_recv,
    rsb_send,
    rsb_recv,
    agf_send,
    agf_recv,
    agb_send,
    agb_recv,
    ax_send,
    ax_recv,
    rsf_credit,
    rsb_credit,
    agf_credit,
    agb_credit,
):
    me = lax.axis_index("i").astype(jnp.int32)
    p = _ring_perm(me)
    nxt = _ring_perm((p + 1) % N_DEV)
    prv = _ring_perm((p - 1) % N_DEV)

    def gemm_chunk(j):
        c = (p - j) % N_DEV
        part_ref[j] = jnp.dot(
            x_ref[pl.ds(c * CH, CH), :].astype(jnp.bfloat16), w_ref[:, :],
            preferred_element_type=jnp.float32,
        ).astype(jnp.bfloat16)

    def rs_desc(dirn, slot, q):
        snd, rcv, ssem, rsem, tgt = (
            (sndf_buf, rsf_buf, rsf_send, rsf_recv, nxt)
            if dirn == 0
            else (sndb_buf, rsb_buf, rsb_send, rsb_recv, prv)
        )
        return pltpu.make_async_remote_copy(
            src_ref=snd.at[slot, q],
            dst_ref=rcv.at[slot, q],
            send_sem=ssem.at[slot, q],
            recv_sem=rsem.at[slot, q],
            device_id=(tgt,),
            device_id_type=pl.DeviceIdType.MESH,
        )

    def ag_desc(dirn, slot, q, t0_src=False):
        buf, ssem, rsem, tgt = (
            (agf_buf, agf_send, agf_recv, nxt)
            if dirn == 0
            else (agb_buf, agb_send, agb_recv, prv)
        )
        src = (qf_ref if dirn == 0 else qb_ref).at[q] if t0_src else buf.at[
            (slot + 1) % 2, q
        ]
        return pltpu.make_async_remote_copy(
            src_ref=src,
            dst_ref=buf.at[slot, q],
            send_sem=ssem.at[slot, q],
            recv_sem=rsem.at[slot, q],
            device_id=(tgt,),
            device_id_type=pl.DeviceIdType.MESH,
        )

    def credit_sig(sem, tgt):
        pl.semaphore_signal(
            sem, inc=1, device_id=(tgt,), device_id_type=pl.DeviceIdType.MESH
        )

    gemm_chunk(0)

    barrier = pltpu.get_barrier_semaphore()
    for nbr in (nxt, prv):
        pl.semaphore_signal(
            barrier, inc=1, device_id=(nbr,), device_id_type=pl.DeviceIdType.MESH
        )
    pl.semaphore_wait(barrier, 2)

    computed = {0}
    for s in range(N_DEV - 1):
        slot, pslot = s % 2, (s - 1) % 2
        for q in range(2):
            for dirn in range(2):
                snd = sndf_buf if dirn == 0 else sndb_buf
                off = q * NQ if dirn == 0 else NH + q * NQ
                if s >= 1:
                    rs_desc(dirn, pslot, q).wait_recv()
                if s >= 2:
                    rs_desc(dirn, slot, q).wait_send()
                if s == 0:
                    snd[slot, q] = part_ref[0, :, off : off + NQ]
                else:
                    pj = s if dirn == 0 else (N_DEV - s) % N_DEV
                    rcv = rsf_buf if dirn == 0 else rsb_buf
                    acc = rcv[pslot, q].astype(jnp.float32) + part_ref[
                        pj, :, off : off + NQ
                    ].astype(jnp.float32)
                    snd[slot, q] = acc.astype(jnp.bfloat16)
                if s >= 2:
                    pl.semaphore_wait(rsf_credit if dirn == 0 else rsb_credit, 1)
                rs_desc(dirn, slot, q).start()
                if 1 <= s <= N_DEV - 3:
                    credit_sig(
                        rsf_credit if dirn == 0 else rsb_credit,
                        prv if dirn == 0 else nxt,
                    )
        for cj in (s + 1, (N_DEV - (s + 1)) % N_DEV):
            if 0 < cj < N_DEV and cj not in computed:
                computed.add(cj)
                gemm_chunk(cj)

    ys = {}
    for q in range(2):
        for dirn in range(2):
            rs_desc(dirn, 0, q).wait_recv()
            pj = N_DEV - 1 if dirn == 0 else 1
            off = q * NQ if dirn == 0 else NH + q * NQ
            rcv = rsf_buf if dirn == 0 else rsb_buf
            y = rcv[0, q].astype(jnp.float32) + part_ref[
                pj, :, off : off + NQ
            ].astype(jnp.float32)
            ys[(dirn, q)] = jnp.maximum(y, 0.0)

    lmax = jnp.maximum(
        jnp.maximum(jnp.max(ys[(0, 0)]), jnp.max(ys[(0, 1)])),
        jnp.maximum(jnp.max(ys[(1, 0)]), jnp.max(ys[(1, 1)])),
    )
    amax_ref[pl.ds(p, 1)] = jnp.full((1, 8, 128), lmax, dtype=jnp.float32)
    ax_sends = []
    for o in range(1, N_DEV):
        tid = _ring_perm((p + o) % N_DEV)
        r = pltpu.make_async_remote_copy(
            src_ref=amax_ref.at[pl.ds(p, 1)],
            dst_ref=amax_ref.at[pl.ds(p, 1)],
            send_sem=ax_send.at[o - 1],
            recv_sem=ax_recv.at[o - 1],
            device_id=(tid,),
            device_id_type=pl.DeviceIdType.MESH,
        )
        r.start()
        ax_sends.append(r)
    for slot_d in (1, 0):
        for q in range(2):
            for dirn in range(2):
                rs_desc(dirn, slot_d, q).wait_send()
    for o in range(1, N_DEV):
        sp = (p - o) % N_DEV
        r = pltpu.make_async_remote_copy(
            src_ref=amax_ref.at[pl.ds(p, 1)],
            dst_ref=amax_ref.at[pl.ds(sp, 1)],
            send_sem=ax_send.at[o - 1],
            recv_sem=ax_recv.at[o - 1],
            device_id=(nxt,),
            device_id_type=pl.DeviceIdType.MESH,
        )
        r.wait_recv()
    for r in ax_sends:
        r.wait_send()

    gmax = jnp.max(amax_ref[:, :, :])
    scale = gmax / 127.0

    own_f = (p + 1) % N_DEV
    own_b = (p - 1) % N_DEV
    t0_descs = []
    for q in range(2):
        qf_ref[q] = jnp.clip(jnp.round(ys[(0, q)] / scale), 0.0, 127.0).astype(
            jnp.int8
        )
        qb_ref[q] = jnp.clip(jnp.round(ys[(1, q)] / scale), 0.0, 127.0).astype(
            jnp.int8
        )
        for dirn in range(2):
            d = ag_desc(dirn, 0, q, t0_src=True)
            d.start()
            t0_descs.append(d)
    for q in range(2):
        out_ref[pl.ds(own_f * CH, CH), q * NQ : (q + 1) * NQ] = (
            qf_ref[q].astype(jnp.float32) * scale
        ).astype(jnp.bfloat16)
        out_ref[pl.ds(own_b * CH, CH), NH + q * NQ : NH + (q + 1) * NQ] = (
            qb_ref[q].astype(jnp.float32) * scale
        ).astype(jnp.bfloat16)
    for d in t0_descs:
        d.wait_send()

    def deq_store(dirn, slot, q, t):
        c = (p - t) % N_DEV if dirn == 0 else (p + t) % N_DEV
        off = q * NQ if dirn == 0 else NH + q * NQ
        buf = agf_buf if dirn == 0 else agb_buf
        out_ref[pl.ds(c * CH, CH), off : off + NQ] = (
            buf[slot, q].astype(jnp.float32) * scale
        ).astype(jnp.bfloat16)

    for t in range(1, N_DEV - 1):
        slot, pslot = t % 2, (t - 1) % 2
        descs = []
        for q in range(2):
            for dirn in range(2):
                ag_desc(dirn, pslot, q).wait_recv()
                if t >= 2:
                    pl.semaphore_wait(agf_credit if dirn == 0 else agb_credit, 1)
                d = ag_desc(dirn, slot, q)
                d.start()
                descs.append((d, dirn, q))
        for d, dirn, q in descs:
            deq_store(dirn, pslot, q, t - 1)
            d.wait_send()
            if t <= N_DEV - 3:
                credit_sig(
                    agf_credit if dirn == 0 else agb_credit,
                    prv if dirn == 0 else nxt,
                )
    for q in range(2):
        for dirn in range(2):
            ag_desc(dirn, 0, q).wait_recv()
            deq_store(dirn, 0, q, N_DEV - 2)


def kernel(x, w_mat):
    w_mat = w_mat.astype(jnp.bfloat16)
    return pl.pallas_call(
        _body,
        out_shape=jax.ShapeDtypeStruct((M, N), jnp.bfloat16),
        in_specs=[
            pl.BlockSpec(memory_space=pltpu.VMEM),
            pl.BlockSpec(memory_space=pltpu.VMEM),
        ],
        out_specs=pl.BlockSpec(memory_space=pltpu.VMEM),
        scratch_shapes=[
            pltpu.VMEM((N_DEV, CH, N), jnp.bfloat16),
            pltpu.VMEM((2, 2, CH, NQ), jnp.bfloat16),
            pltpu.VMEM((2, 2, CH, NQ), jnp.bfloat16),
            pltpu.VMEM((2, 2, CH, NQ), jnp.bfloat16),
            pltpu.VMEM((2, 2, CH, NQ), jnp.bfloat16),
            pltpu.VMEM((2, 2, CH, NQ), jnp.int8),
            pltpu.VMEM((2, 2, CH, NQ), jnp.int8),
            pltpu.VMEM((2, CH, NQ), jnp.int8),
            pltpu.VMEM((2, CH, NQ), jnp.int8),
            pltpu.VMEM((N_DEV, 8, 128), jnp.float32),
            pltpu.SemaphoreType.DMA((2, 2)),
            pltpu.SemaphoreType.DMA((2, 2)),
            pltpu.SemaphoreType.DMA((2, 2)),
            pltpu.SemaphoreType.DMA((2, 2)),
            pltpu.SemaphoreType.DMA((2, 2)),
            pltpu.SemaphoreType.DMA((2, 2)),
            pltpu.SemaphoreType.DMA((2, 2)),
            pltpu.SemaphoreType.DMA((2, 2)),
            pltpu.SemaphoreType.DMA((N_DEV - 1,)),
            pltpu.SemaphoreType.DMA((N_DEV - 1,)),
            pltpu.SemaphoreType.REGULAR,
            pltpu.SemaphoreType.REGULAR,
            pltpu.SemaphoreType.REGULAR,
            pltpu.SemaphoreType.REGULAR,
        ],
        compiler_params=pltpu.CompilerParams(
            collective_id=0, vmem_limit_bytes=60 * 1024 * 1024
        ),
    )(x, w_mat)


# device time: 155102 ns/iter; 2.0755x vs baseline; 1.0505x over previous
import jax
import jax.numpy as jnp
from jax import lax
from jax.experimental import pallas as pl
from jax.experimental.pallas import tpu as pltpu

N_DEV = 8
M, K_SH, N = 4096, 512, 2048
CH = M // N_DEV
NH = N // 2
NQ = NH // 2


def _ring_perm(v):
    v = jnp.where(v == 4, 7, jnp.where(v == 7, 4, v))
    return jnp.where(v == 5, 6, jnp.where(v == 6, 5, v))


def _body(
    x_ref,
    w_ref,
    out_ref,
    wbf_ref,
    ostage,
    ocopy_sem,
    part_ref,
    rsf_buf,
    rsb_buf,
    sndf_buf,
    sndb_buf,
    agf_buf,
    agb_buf,
    qf_ref,
    qb_ref,
    amax_ref,
    rsf_send,
    rsf_recv,
    rsb_send,
    rsb_recv,
    agf_send,
    agf_recv,
    agb_send,
    agb_recv,
    ax_send,
    ax_recv,
    rsf_credit,
    rsb_credit,
    agf_credit,
    agb_credit,
):
    me = lax.axis_index("i").astype(jnp.int32)
    p = _ring_perm(me)
    nxt = _ring_perm((p + 1) % N_DEV)
    prv = _ring_perm((p - 1) % N_DEV)

    def gemm_chunk(j):
        c = (p - j) % N_DEV
        part_ref[j] = jnp.dot(
            x_ref[pl.ds(c * CH, CH), :].astype(jnp.bfloat16), wbf_ref[:, :],
            preferred_element_type=jnp.float32,
        ).astype(jnp.bfloat16)

    ostage_desc = {}

    def deq_to_out(dirn, q, val, c, off):
        key = (dirn, q)
        if key in ostage_desc:
            ostage_desc[key].wait()
        ostage[dirn, q] = val
        d = pltpu.make_async_copy(
            ostage.at[dirn, q],
            out_ref.at[pl.ds(c * CH, CH), pl.ds(off, NQ)],
            ocopy_sem.at[dirn, q],
        )
        d.start()
        ostage_desc[key] = d

    def rs_desc(dirn, slot, q):
        snd, rcv, ssem, rsem, tgt = (
            (sndf_buf, rsf_buf, rsf_send, rsf_recv, nxt)
            if dirn == 0
            else (sndb_buf, rsb_buf, rsb_send, rsb_recv, prv)
        )
        return pltpu.make_async_remote_copy(
            src_ref=snd.at[slot, q],
            dst_ref=rcv.at[slot, q],
            send_sem=ssem.at[slot, q],
            recv_sem=rsem.at[slot, q],
            device_id=(tgt,),
            device_id_type=pl.DeviceIdType.MESH,
        )

    def ag_desc(dirn, slot, q, t0_src=False):
        buf, ssem, rsem, tgt = (
            (agf_buf, agf_send, agf_recv, nxt)
            if dirn == 0
            else (agb_buf, agb_send, agb_recv, prv)
        )
        src = (qf_ref if dirn == 0 else qb_ref).at[q] if t0_src else buf.at[
            (slot + 1) % 2, q
        ]
        return pltpu.make_async_remote_copy(
            src_ref=src,
            dst_ref=buf.at[slot, q],
            send_sem=ssem.at[slot, q],
            recv_sem=rsem.at[slot, q],
            device_id=(tgt,),
            device_id_type=pl.DeviceIdType.MESH,
        )

    def credit_sig(sem, tgt):
        pl.semaphore_signal(
            sem, inc=1, device_id=(tgt,), device_id_type=pl.DeviceIdType.MESH
        )

    wbf_ref[:, :] = w_ref[:, :].astype(jnp.bfloat16)
    gemm_chunk(0)

    barrier = pltpu.get_barrier_semaphore()
    for nbr in (nxt, prv):
        pl.semaphore_signal(
            barrier, inc=1, device_id=(nbr,), device_id_type=pl.DeviceIdType.MESH
        )
    pl.semaphore_wait(barrier, 2)

    computed = {0}
    for s in range(N_DEV - 1):
        slot, pslot = s % 2, (s - 1) % 2
        for q in range(2):
            for dirn in range(2):
                snd = sndf_buf if dirn == 0 else sndb_buf
                off = q * NQ if dirn == 0 else NH + q * NQ
                if s >= 1:
                    rs_desc(dirn, pslot, q).wait_recv()
                if s >= 2:
                    rs_desc(dirn, slot, q).wait_send()
                if s == 0:
                    snd[slot, q] = part_ref[0, :, off : off + NQ]
                else:
                    pj = s if dirn == 0 else (N_DEV - s) % N_DEV
                    rcv = rsf_buf if dirn == 0 else rsb_buf
                    acc = rcv[pslot, q].astype(jnp.float32) + part_ref[
                        pj, :, off : off + NQ
                    ].astype(jnp.float32)
                    snd[slot, q] = acc.astype(jnp.bfloat16)
                if s >= 2:
                    pl.semaphore_wait(rsf_credit if dirn == 0 else rsb_credit, 1)
                rs_desc(dirn, slot, q).start()
                if 1 <= s <= N_DEV - 3:
                    credit_sig(
                        rsf_credit if dirn == 0 else rsb_credit,
                        prv if dirn == 0 else nxt,
                    )
        for cj in (s + 1, (N_DEV - (s + 1)) % N_DEV):
            if 0 < cj < N_DEV and cj not in computed:
                computed.add(cj)
                gemm_chunk(cj)

    ys = {}
    for q in range(2):
        for dirn in range(2):
            rs_desc(dirn, 0, q).wait_recv()
            pj = N_DEV - 1 if dirn == 0 else 1
            off = q * NQ if dirn == 0 else NH + q * NQ
            rcv = rsf_buf if dirn == 0 else rsb_buf
            y = rcv[0, q].astype(jnp.float32) + part_ref[
                pj, :, off : off + NQ
            ].astype(jnp.float32)
            ys[(dirn, q)] = jnp.maximum(y, 0.0)

    lmax = jnp.maximum(
        jnp.maximum(jnp.max(ys[(0, 0)]), jnp.max(ys[(0, 1)])),
        jnp.maximum(jnp.max(ys[(1, 0)]), jnp.max(ys[(1, 1)])),
    )
    amax_ref[pl.ds(p, 1)] = jnp.full((1, 8, 128), lmax, dtype=jnp.float32)
    ax_sends = []
    for o in range(1, N_DEV):
        tid = _ring_perm((p + o) % N_DEV)
        r = pltpu.make_async_remote_copy(
            src_ref=amax_ref.at[pl.ds(p, 1)],
            dst_ref=amax_ref.at[pl.ds(p, 1)],
            send_sem=ax_send.at[o - 1],
            recv_sem=ax_recv.at[o - 1],
            device_id=(tid,),
            device_id_type=pl.DeviceIdType.MESH,
        )
        r.start()
        ax_sends.append(r)
    for slot_d in (1, 0):
        for q in range(2):
            for dirn in range(2):
                rs_desc(dirn, slot_d, q).wait_send()
    for o in range(1, N_DEV):
        sp = (p - o) % N_DEV
        r = pltpu.make_async_remote_copy(
            src_ref=amax_ref.at[pl.ds(p, 1)],
            dst_ref=amax_ref.at[pl.ds(sp, 1)],
            send_sem=ax_send.at[o - 1],
            recv_sem=ax_recv.at[o - 1],
            device_id=(nxt,),
            device_id_type=pl.DeviceIdType.MESH,
        )
        r.wait_recv()
    for r in ax_sends:
        r.wait_send()

    gmax = jnp.max(amax_ref[:, :, :])
    scale = gmax / 127.0

    own_f = (p + 1) % N_DEV
    own_b = (p - 1) % N_DEV
    t0_descs = []
    for q in range(2):
        qf_ref[q] = jnp.clip(jnp.round(ys[(0, q)] / scale), 0.0, 127.0).astype(
            jnp.int8
        )
        qb_ref[q] = jnp.clip(jnp.round(ys[(1, q)] / scale), 0.0, 127.0).astype(
            jnp.int8
        )
        for dirn in range(2):
            d = ag_desc(dirn, 0, q, t0_src=True)
            d.start()
            t0_descs.append(d)
    for q in range(2):
        deq_to_out(
            0,
            q,
            (qf_ref[q].astype(jnp.float32) * scale).astype(jnp.bfloat16),
            own_f,
            q * NQ,
        )
        deq_to_out(
            1,
            q,
            (qb_ref[q].astype(jnp.float32) * scale).astype(jnp.bfloat16),
            own_b,
            NH + q * NQ,
        )
    for d in t0_descs:
        d.wait_send()

    def deq_store(dirn, slot, q, t):
        c = (p - t) % N_DEV if dirn == 0 else (p + t) % N_DEV
        off = q * NQ if dirn == 0 else NH + q * NQ
        buf = agf_buf if dirn == 0 else agb_buf
        deq_to_out(
            dirn,
            q,
            (buf[slot, q].astype(jnp.float32) * scale).astype(jnp.bfloat16),
            c,
            off,
        )

    for t in range(1, N_DEV - 1):
        slot, pslot = t % 2, (t - 1) % 2
        descs = []
        for q in range(2):
            for dirn in range(2):
                ag_desc(dirn, pslot, q).wait_recv()
                if t >= 2:
                    pl.semaphore_wait(agf_credit if dirn == 0 else agb_credit, 1)
                d = ag_desc(dirn, slot, q)
                d.start()
                descs.append((d, dirn, q))
        for d, dirn, q in descs:
            deq_store(dirn, pslot, q, t - 1)
            d.wait_send()
            if t <= N_DEV - 3:
                credit_sig(
                    agf_credit if dirn == 0 else agb_credit,
                    prv if dirn == 0 else nxt,
                )
    for q in range(2):
        for dirn in range(2):
            ag_desc(dirn, 0, q).wait_recv()
            deq_store(dirn, 0, q, N_DEV - 2)
    for d in ostage_desc.values():
        d.wait()


def kernel(x, w_mat):
    return pl.pallas_call(
        _body,
        out_shape=jax.ShapeDtypeStruct((M, N), jnp.bfloat16),
        in_specs=[
            pl.BlockSpec(memory_space=pltpu.VMEM),
            pl.BlockSpec(memory_space=pltpu.VMEM),
        ],
        out_specs=pl.BlockSpec(memory_space=pl.ANY),
        scratch_shapes=[
            pltpu.VMEM((K_SH, N), jnp.bfloat16),
            pltpu.VMEM((2, 2, CH, NQ), jnp.bfloat16),
            pltpu.SemaphoreType.DMA((2, 2)),
            pltpu.VMEM((N_DEV, CH, N), jnp.bfloat16),
            pltpu.VMEM((2, 2, CH, NQ), jnp.bfloat16),
            pltpu.VMEM((2, 2, CH, NQ), jnp.bfloat16),
            pltpu.VMEM((2, 2, CH, NQ), jnp.bfloat16),
            pltpu.VMEM((2, 2, CH, NQ), jnp.bfloat16),
            pltpu.VMEM((2, 2, CH, NQ), jnp.int8),
            pltpu.VMEM((2, 2, CH, NQ), jnp.int8),
            pltpu.VMEM((2, CH, NQ), jnp.int8),
            pltpu.VMEM((2, CH, NQ), jnp.int8),
            pltpu.VMEM((N_DEV, 8, 128), jnp.float32),
            pltpu.SemaphoreType.DMA((2, 2)),
            pltpu.SemaphoreType.DMA((2, 2)),
            pltpu.SemaphoreType.DMA((2, 2)),
            pltpu.SemaphoreType.DMA((2, 2)),
            pltpu.SemaphoreType.DMA((2, 2)),
            pltpu.SemaphoreType.DMA((2, 2)),
            pltpu.SemaphoreType.DMA((2, 2)),
            pltpu.SemaphoreType.DMA((2, 2)),
            pltpu.SemaphoreType.DMA((N_DEV - 1,)),
            pltpu.SemaphoreType.DMA((N_DEV - 1,)),
            pltpu.SemaphoreType.REGULAR,
            pltpu.SemaphoreType.REGULAR,
            pltpu.SemaphoreType.REGULAR,
            pltpu.SemaphoreType.REGULAR,
        ],
        compiler_params=pltpu.CompilerParams(
            collective_id=0, vmem_limit_bytes=60 * 1024 * 1024
        ),
    )(x, w_mat)
